# baseline (speedup 1.0000x reference)
import jax
import jax.numpy as jnp
from jax import lax
from jax.experimental import pallas as pl
from jax.experimental.pallas import tpu as pltpu
from jax.experimental.pallas import tpu_sc as plsc

_N = 2048
_L = 16
_S = _N // _L
_NBITS = 8
_NBINS = 1 << _NBITS
_HV = _NBINS // _L
_PASSES = 4
_B = 4
_K = 4
_MINI = -(1 << 31)


def _fspool_body(num_workers, chans_per_worker, core_axis, subcore_axis):
    def body(x_hbm, wflip_hbm, out_hbm, kbufA, kbufB, hist, xstage, wbuf2,
             accbuf, resbuf, sumbuf, basebuf, sem):
        bufs = (kbufA, kbufB)

        wid = lax.axis_index(subcore_axis) * 2 + lax.axis_index(core_axis)
        c0 = wid * chans_per_worker

        for r in range(_B):
            pltpu.async_copy(x_hbm.at[r, c0], xstage.at[pl.ds(r * _N, _N)],
                             sem)
        pltpu.async_copy(wflip_hbm.at[c0], wbuf2.at[pl.ds(0, _N)], sem)

        def chan_body(ci, _):
            c = c0 + ci
            par = (ci & 1) * (_B * _N)
            wpar = (ci & 1) * _N
            for r in range(_B):
                pltpu.make_async_copy(
                    x_hbm.at[r, c], xstage.at[pl.ds(par + r * _N, _N)],
                    sem).wait()
            pltpu.make_async_copy(wflip_hbm.at[c],
                                  wbuf2.at[pl.ds(wpar, _N)], sem).wait()

            @pl.when(ci + 1 < chans_per_worker)
            def _():
                npar = ((ci + 1) & 1) * (_B * _N)
                nwpar = ((ci + 1) & 1) * _N
                for r in range(_B):
                    pltpu.async_copy(x_hbm.at[r, c + 1],
                                     xstage.at[pl.ds(npar + r * _N, _N)],
                                     sem)
                pltpu.async_copy(wflip_hbm.at[c + 1],
                                 wbuf2.at[pl.ds(nwpar, _N)], sem)

            z = jnp.zeros((_L,), jnp.float32)
            for r in range(_B):
                accbuf[pl.ds(r * _L, _L)] = z

            @plsc.parallel_loop(0, _B * _PASSES * _HV, unroll=4)
            def _clr(i):
                hist[pl.ds(i * _L, _L)] = jnp.zeros((_L,), jnp.int32)

            @plsc.parallel_loop(0, _S, unroll=2)
            def _prep(s):
                base = s * _L
                ones = jnp.ones((_L,), jnp.int32)
                for r in range(_B):
                    v = xstage[pl.ds(par + r * _N + base, _L)]
                    u = plsc.bitcast(v, jnp.int32)
                    m = lax.shift_right_arithmetic(u, 31)
                    key = u ^ (m | _MINI)
                    kbufA[pl.ds(r * _N + base, _L)] = \
                        plsc.bitcast(key, jnp.float32)
                    for p in range(_PASSES):
                        d = lax.shift_right_logical(key, p * _NBITS) \
                            & (_NBINS - 1)
                        plsc.addupdate_scatter(
                            hist, [d + (r * _PASSES + p) * _NBINS], ones)

            for p in range(_PASSES):
                sh = p * _NBITS
                src = bufs[p % 2]
                dst = bufs[1 - p % 2]
                final = p == _PASSES - 1

                lane0 = lax.iota(jnp.int32, _L) == 0

                @plsc.parallel_loop(0, _B * _HV, unroll=2)
                def _sums(j, p=p):
                    hb = ((j >> 4) * _PASSES + p) * _NBINS + (j & 15) * _L
                    v = hist[pl.ds(hb, _L)]
                    s = jnp.sum(v)
                    plsc.store_scatter(sumbuf, [jnp.full((_L,), j,
                                                         jnp.int32)],
                                       jnp.full((_L,), s), mask=lane0)

                for r in range(_B):
                    sv = sumbuf[pl.ds(r * _HV, _HV)]
                    basebuf[pl.ds(r * _HV, _HV)] =                         plsc.cumsum(sv) - sv + jnp.int32(-1)

                @plsc.parallel_loop(0, _B * _HV, unroll=2)
                def _excl(j, p=p):
                    hb = ((j >> 4) * _PASSES + p) * _NBINS + (j & 15) * _L
                    v = hist[pl.ds(hb, _L)]
                    b = plsc.load_gather(basebuf,
                                         [jnp.full((_L,), j, jnp.int32)])
                    hist[pl.ds(hb, _L)] = plsc.cumsum(v) - v + b

                if not final:
                    def perm_body(s, _, src=src, dst=dst, p=p, sh=sh):
                        base = s * (_K * _L)

                        @plsc.parallel_loop(0, _B, unroll=_B)
                        def _rows(r):
                            hb = (r * _PASSES + p) * _NBINS
                            rbase = r * _N + base
                            vs, ds_, cs, ls = [], [], [], []
                            for k in range(_K):
                                v = src[pl.ds(rbase + k * _L, _L)]
                                u = plsc.bitcast(v, jnp.int32)
                                d = (lax.shift_right_logical(u, sh)
                                     & (_NBINS - 1)) + hb
                                cnt, last = plsc.scan_count(d)
                                vs.append(v)
                                ds_.append(d)
                                cs.append(cnt)
                                ls.append(last)
                            for k in range(_K):
                                pos = plsc.load_gather(hist, [ds_[k]]) + cs[k]
                                plsc.addupdate_scatter(hist, [ds_[k]], cs[k],
                                                       mask=ls[k])
                                plsc.store_scatter(dst, [pos + r * _N], vs[k])
                        return 0

                    lax.fori_loop(0, _S // _K, perm_body, 0)
                else:
                    def final_body(s, _, src=src, p=p, sh=sh):
                        base = s * (_K * _L)

                        @plsc.parallel_loop(0, _B, unroll=_B)
                        def _rows(r):
                            hb = (r * _PASSES + p) * _NBINS
                            rbase = r * _N + base
                            us, ds_, cs, ls = [], [], [], []
                            for k in range(_K):
                                v = src[pl.ds(rbase + k * _L, _L)]
                                u = plsc.bitcast(v, jnp.int32)
                                d = (lax.shift_right_logical(u, sh)
                                     & (_NBINS - 1)) + hb
                                cnt, last = plsc.scan_count(d)
                                us.append(u)
                                ds_.append(d)
                                cs.append(cnt)
                                ls.append(last)
                            acc = None
                            for k in range(_K):
                                pos = plsc.load_gather(hist, [ds_[k]]) + cs[k]
                                plsc.addupdate_scatter(hist, [ds_[k]], cs[k],
                                                       mask=ls[k])
                                wv = plsc.load_gather(wbuf2, [pos + wpar])
                                m2 = lax.shift_right_arithmetic(us[k], 31)
                                og = us[k] ^ (~m2 | _MINI)
                                term = plsc.bitcast(og, jnp.float32) * wv
                                acc = term if acc is None else acc + term
                            plsc.addupdate(accbuf.at[pl.ds(r * _L, _L)], acc)
                        return 0

                    lax.fori_loop(0, _S // _K, final_body, 0)

            lane0b = lax.iota(jnp.int32, _L) == 0
            for r in range(_B):
                res = jnp.sum(accbuf[pl.ds(r * _L, _L)])
                idx = jnp.full((_L,), r * chans_per_worker + ci, jnp.int32)
                plsc.store_scatter(resbuf, [idx], jnp.full((_L,), res),
                                   mask=lane0b)
            return 0

        lax.fori_loop(0, chans_per_worker, chan_body, 0)

        for r in range(_B):
            pltpu.sync_copy(
                resbuf.at[pl.ds(r * chans_per_worker, chans_per_worker)],
                out_hbm.at[r, pl.ds(c0, chans_per_worker)])

    return body


def kernel(x, weight):
    b, c, n = x.shape
    assert (b, n) == (_B, _N) and weight.shape == (c, n)
    info = plsc.get_sparse_core_info()
    num_workers = info.num_cores * info.num_subcores
    chans_per_worker = c // num_workers
    mesh = plsc.VectorSubcoreMesh(core_axis_name="sc_core",
                                  subcore_axis_name="sc_subcore")
    scratch = [
        pltpu.VMEM((_B * _N,), jnp.float32),
        pltpu.VMEM((_B * _N,), jnp.float32),
        pltpu.VMEM((_B * _PASSES * _NBINS,), jnp.int32),
        pltpu.VMEM((2 * _B * _N,), jnp.float32),
        pltpu.VMEM((2 * _N,), jnp.float32),
        pltpu.VMEM((_B * _L,), jnp.float32),
        pltpu.VMEM((_B * chans_per_worker,), jnp.float32),
        pltpu.VMEM((_B * _HV,), jnp.int32),
        pltpu.VMEM((_B * _HV,), jnp.int32),
        pltpu.SemaphoreType.DMA,
    ]
    k = pl.kernel(
        _fspool_body(num_workers, chans_per_worker, "sc_core", "sc_subcore"),
        out_type=jax.ShapeDtypeStruct((b, c), jnp.float32),
        mesh=mesh,
        scratch_types=scratch,
        compiler_params=pltpu.CompilerParams(needs_layout_passes=False),
    )
    wflip = jnp.flip(weight, axis=1)
    return k(x, wflip)


# speedup vs baseline: 1.0343x; 1.0343x over previous
import jax
import jax.numpy as jnp
from jax import lax
from jax.experimental import pallas as pl
from jax.experimental.pallas import tpu as pltpu
from jax.experimental.pallas import tpu_sc as plsc

_N = 2048
_L = 16
_S = _N // _L
_NBITS = 8
_NBINS = 1 << _NBITS
_HV = _NBINS // _L
_PASSES = 4
_B = 4
_K = 4
_MINI = -(1 << 31)


def _fspool_body(num_workers, chans_per_worker, core_axis, subcore_axis):
    def body(x_hbm, wflip_hbm, out_hbm, kbufA, kbufB, hist, xstage, wbuf2,
             accbuf, resbuf, sem):
        bufs = (kbufA, kbufB)

        wid = lax.axis_index(subcore_axis) * 2 + lax.axis_index(core_axis)
        c0 = wid * chans_per_worker

        for r in range(_B):
            pltpu.async_copy(x_hbm.at[r, c0], xstage.at[pl.ds(r * _N, _N)],
                             sem)
        pltpu.async_copy(wflip_hbm.at[c0], wbuf2.at[pl.ds(0, _N)], sem)

        def chan_body(ci, _):
            c = c0 + ci
            par = (ci & 1) * (_B * _N)
            wpar = (ci & 1) * _N
            for r in range(_B):
                pltpu.make_async_copy(
                    x_hbm.at[r, c], xstage.at[pl.ds(par + r * _N, _N)],
                    sem).wait()
            pltpu.make_async_copy(wflip_hbm.at[c],
                                  wbuf2.at[pl.ds(wpar, _N)], sem).wait()

            @pl.when(ci + 1 < chans_per_worker)
            def _():
                npar = ((ci + 1) & 1) * (_B * _N)
                nwpar = ((ci + 1) & 1) * _N
                for r in range(_B):
                    pltpu.async_copy(x_hbm.at[r, c + 1],
                                     xstage.at[pl.ds(npar + r * _N, _N)],
                                     sem)
                pltpu.async_copy(wflip_hbm.at[c + 1],
                                 wbuf2.at[pl.ds(nwpar, _N)], sem)

            z = jnp.zeros((_L,), jnp.float32)
            for r in range(_B):
                accbuf[pl.ds(r * _L, _L)] = z

            @plsc.parallel_loop(0, _B * _PASSES * _HV, unroll=4)
            def _clr(i):
                hist[pl.ds(i * _L, _L)] = jnp.zeros((_L,), jnp.int32)

            @plsc.parallel_loop(0, _S, unroll=2)
            def _prep(s):
                base = s * _L
                ones = jnp.ones((_L,), jnp.int32)
                for r in range(_B):
                    v = xstage[pl.ds(par + r * _N + base, _L)]
                    u = plsc.bitcast(v, jnp.int32)
                    m = lax.shift_right_arithmetic(u, 31)
                    key = u ^ (m | _MINI)
                    kbufA[pl.ds(r * _N + base, _L)] = \
                        plsc.bitcast(key, jnp.float32)
                    for p in range(_PASSES):
                        d = lax.shift_right_logical(key, p * _NBITS) \
                            & (_NBINS - 1)
                        plsc.addupdate_scatter(
                            hist, [d + (r * _PASSES + p) * _NBINS], ones)

            for p in range(_PASSES):
                sh = p * _NBITS
                src = bufs[p % 2]
                dst = bufs[1 - p % 2]
                final = p == _PASSES - 1

                def scan_body(i, carry, p=p):
                    out = []
                    for r in range(_B):
                        hb = (r * _PASSES + p) * _NBINS + i * _L
                        v = hist[pl.ds(hb, _L)]
                        inc = plsc.cumsum(v)
                        hist[pl.ds(hb, _L)] = inc - v + carry[r]
                        out.append(carry[r] + jnp.sum(v))
                    return tuple(out)

                lax.fori_loop(0, _HV, scan_body, (jnp.int32(-1),) * _B,
                              unroll=2)

                if not final:
                    def perm_body(s, _, src=src, dst=dst, p=p, sh=sh):
                        base = s * (_K * _L)

                        @plsc.parallel_loop(0, _B, unroll=_B)
                        def _rows(r):
                            hb = (r * _PASSES + p) * _NBINS
                            rbase = r * _N + base
                            vs, ds_, cs, ls = [], [], [], []
                            for k in range(_K):
                                v = src[pl.ds(rbase + k * _L, _L)]
                                u = plsc.bitcast(v, jnp.int32)
                                d = (lax.shift_right_logical(u, sh)
                                     & (_NBINS - 1)) + hb
                                cnt, last = plsc.scan_count(d)
                                vs.append(v)
                                ds_.append(d)
                                cs.append(cnt)
                                ls.append(last)
                            for k in range(_K):
                                pos = plsc.load_gather(hist, [ds_[k]]) + cs[k]
                                plsc.addupdate_scatter(hist, [ds_[k]], cs[k],
                                                       mask=ls[k])
                                plsc.store_scatter(dst, [pos + r * _N], vs[k])
                        return 0

                    lax.fori_loop(0, _S // _K, perm_body, 0)
                else:
                    def final_body(s, _, src=src, p=p, sh=sh):
                        base = s * (_K * _L)

                        @plsc.parallel_loop(0, _B, unroll=_B)
                        def _rows(r):
                            hb = (r * _PASSES + p) * _NBINS
                            rbase = r * _N + base
                            us, ds_, cs, ls = [], [], [], []
                            for k in range(_K):
                                v = src[pl.ds(rbase + k * _L, _L)]
                                u = plsc.bitcast(v, jnp.int32)
                                d = (lax.shift_right_logical(u, sh)
                                     & (_NBINS - 1)) + hb
                                cnt, last = plsc.scan_count(d)
                                us.append(u)
                                ds_.append(d)
                                cs.append(cnt)
                                ls.append(last)
                            acc = None
                            for k in range(_K):
                                pos = plsc.load_gather(hist, [ds_[k]]) + cs[k]
                                plsc.addupdate_scatter(hist, [ds_[k]], cs[k],
                                                       mask=ls[k])
                                wv = plsc.load_gather(wbuf2, [pos + wpar])
                                m2 = lax.shift_right_arithmetic(us[k], 31)
                                og = us[k] ^ (~m2 | _MINI)
                                term = plsc.bitcast(og, jnp.float32) * wv
                                acc = term if acc is None else acc + term
                            plsc.addupdate(accbuf.at[pl.ds(r * _L, _L)], acc)
                        return 0

                    lax.fori_loop(0, _S // _K, final_body, 0)

            lane0 = lax.iota(jnp.int32, _L) == 0
            for r in range(_B):
                res = jnp.sum(accbuf[pl.ds(r * _L, _L)])
                idx = jnp.full((_L,), r * chans_per_worker + ci, jnp.int32)
                plsc.store_scatter(resbuf, [idx], jnp.full((_L,), res),
                                   mask=lane0)
            return 0

        lax.fori_loop(0, chans_per_worker, chan_body, 0)

        for r in range(_B):
            pltpu.sync_copy(
                resbuf.at[pl.ds(r * chans_per_worker, chans_per_worker)],
                out_hbm.at[r, pl.ds(c0, chans_per_worker)])

    return body


def kernel(x, weight):
    b, c, n = x.shape
    assert (b, n) == (_B, _N) and weight.shape == (c, n)
    info = plsc.get_sparse_core_info()
    num_workers = info.num_cores * info.num_subcores
    chans_per_worker = c // num_workers
    mesh = plsc.VectorSubcoreMesh(core_axis_name="sc_core",
                                  subcore_axis_name="sc_subcore")
    scratch = [
        pltpu.VMEM((_B * _N,), jnp.float32),
        pltpu.VMEM((_B * _N,), jnp.float32),
        pltpu.VMEM((_B * _PASSES * _NBINS,), jnp.int32),
        pltpu.VMEM((2 * _B * _N,), jnp.float32),
        pltpu.VMEM((2 * _N,), jnp.float32),
        pltpu.VMEM((_B * _L,), jnp.float32),
        pltpu.VMEM((_B * chans_per_worker,), jnp.float32),
        pltpu.SemaphoreType.DMA,
    ]
    k = pl.kernel(
        _fspool_body(num_workers, chans_per_worker, "sc_core", "sc_subcore"),
        out_type=jax.ShapeDtypeStruct((b, c), jnp.float32),
        mesh=mesh,
        scratch_types=scratch,
        compiler_params=pltpu.CompilerParams(needs_layout_passes=False),
    )
    wflip = jnp.flip(weight, axis=1)
    return k(x, wflip)


# speedup vs baseline: 1.0349x; 1.0006x over previous
import jax
import jax.numpy as jnp
from jax import lax
from jax.experimental import pallas as pl
from jax.experimental.pallas import tpu as pltpu
from jax.experimental.pallas import tpu_sc as plsc

_N = 2048
_L = 16
_S = _N // _L
_NBITS = 8
_NBINS = 1 << _NBITS
_HV = _NBINS // _L
_PASSES = 4
_B = 4
_K = 8
_MINI = -(1 << 31)


def _fspool_body(num_workers, chans_per_worker, core_axis, subcore_axis):
    def body(x_hbm, wflip_hbm, out_hbm, kbufA, kbufB, hist, xstage, wbuf2,
             accbuf, resbuf, sem):
        bufs = (kbufA, kbufB)

        wid = lax.axis_index(subcore_axis) * 2 + lax.axis_index(core_axis)
        c0 = wid * chans_per_worker

        for r in range(_B):
            pltpu.async_copy(x_hbm.at[r, c0], xstage.at[pl.ds(r * _N, _N)],
                             sem)
        pltpu.async_copy(wflip_hbm.at[c0], wbuf2.at[pl.ds(0, _N)], sem)

        def chan_body(ci, _):
            c = c0 + ci
            par = (ci & 1) * (_B * _N)
            wpar = (ci & 1) * _N
            for r in range(_B):
                pltpu.make_async_copy(
                    x_hbm.at[r, c], xstage.at[pl.ds(par + r * _N, _N)],
                    sem).wait()
            pltpu.make_async_copy(wflip_hbm.at[c],
                                  wbuf2.at[pl.ds(wpar, _N)], sem).wait()

            @pl.when(ci + 1 < chans_per_worker)
            def _():
                npar = ((ci + 1) & 1) * (_B * _N)
                nwpar = ((ci + 1) & 1) * _N
                for r in range(_B):
                    pltpu.async_copy(x_hbm.at[r, c + 1],
                                     xstage.at[pl.ds(npar + r * _N, _N)],
                                     sem)
                pltpu.async_copy(wflip_hbm.at[c + 1],
                                 wbuf2.at[pl.ds(nwpar, _N)], sem)

            z = jnp.zeros((_L,), jnp.float32)
            for r in range(_B):
                accbuf[pl.ds(r * _L, _L)] = z

            @plsc.parallel_loop(0, _B * _PASSES * _HV, unroll=4)
            def _clr(i):
                hist[pl.ds(i * _L, _L)] = jnp.zeros((_L,), jnp.int32)

            @plsc.parallel_loop(0, _S, unroll=2)
            def _prep(s):
                base = s * _L
                ones = jnp.ones((_L,), jnp.int32)
                for r in range(_B):
                    v = xstage[pl.ds(par + r * _N + base, _L)]
                    u = plsc.bitcast(v, jnp.int32)
                    m = lax.shift_right_arithmetic(u, 31)
                    key = u ^ (m | _MINI)
                    kbufA[pl.ds(r * _N + base, _L)] = \
                        plsc.bitcast(key, jnp.float32)
                    for p in range(_PASSES):
                        d = lax.shift_right_logical(key, p * _NBITS) \
                            & (_NBINS - 1)
                        plsc.addupdate_scatter(
                            hist, [d + (r * _PASSES + p) * _NBINS], ones)

            for p in range(_PASSES):
                sh = p * _NBITS
                src = bufs[p % 2]
                dst = bufs[1 - p % 2]
                final = p == _PASSES - 1

                def scan_body(i, carry, p=p):
                    out = []
                    for r in range(_B):
                        hb = (r * _PASSES + p) * _NBINS + i * _L
                        v = hist[pl.ds(hb, _L)]
                        inc = plsc.cumsum(v)
                        hist[pl.ds(hb, _L)] = inc - v + carry[r]
                        out.append(carry[r] + jnp.sum(v))
                    return tuple(out)

                lax.fori_loop(0, _HV, scan_body, (jnp.int32(-1),) * _B,
                              unroll=2)

                if not final:
                    def perm_body(s, _, src=src, dst=dst, p=p, sh=sh):
                        base = s * (_K * _L)

                        @plsc.parallel_loop(0, _B, unroll=_B)
                        def _rows(r):
                            hb = (r * _PASSES + p) * _NBINS
                            rbase = r * _N + base
                            vs, ds_, cs, ls = [], [], [], []
                            for k in range(_K):
                                v = src[pl.ds(rbase + k * _L, _L)]
                                u = plsc.bitcast(v, jnp.int32)
                                d = (lax.shift_right_logical(u, sh)
                                     & (_NBINS - 1)) + hb
                                cnt, last = plsc.scan_count(d)
                                vs.append(v)
                                ds_.append(d)
                                cs.append(cnt)
                                ls.append(last)
                            for k in range(_K):
                                pos = plsc.load_gather(hist, [ds_[k]]) + cs[k]
                                plsc.addupdate_scatter(hist, [ds_[k]], cs[k],
                                                       mask=ls[k])
                                plsc.store_scatter(dst, [pos + r * _N], vs[k])
                        return 0

                    lax.fori_loop(0, _S // _K, perm_body, 0)
                else:
                    def final_body(s, _, src=src, p=p, sh=sh):
                        base = s * (_K * _L)

                        @plsc.parallel_loop(0, _B, unroll=_B)
                        def _rows(r):
                            hb = (r * _PASSES + p) * _NBINS
                            rbase = r * _N + base
                            us, ds_, cs, ls = [], [], [], []
                            for k in range(_K):
                                v = src[pl.ds(rbase + k * _L, _L)]
                                u = plsc.bitcast(v, jnp.int32)
                                d = (lax.shift_right_logical(u, sh)
                                     & (_NBINS - 1)) + hb
                                cnt, last = plsc.scan_count(d)
                                us.append(u)
                                ds_.append(d)
                                cs.append(cnt)
                                ls.append(last)
                            acc = None
                            for k in range(_K):
                                pos = plsc.load_gather(hist, [ds_[k]]) + cs[k]
                                plsc.addupdate_scatter(hist, [ds_[k]], cs[k],
                                                       mask=ls[k])
                                wv = plsc.load_gather(wbuf2, [pos + wpar])
                                m2 = lax.shift_right_arithmetic(us[k], 31)
                                og = us[k] ^ (~m2 | _MINI)
                                term = plsc.bitcast(og, jnp.float32) * wv
                                acc = term if acc is None else acc + term
                            plsc.addupdate(accbuf.at[pl.ds(r * _L, _L)], acc)
                        return 0

                    lax.fori_loop(0, _S // _K, final_body, 0)

            lane0 = lax.iota(jnp.int32, _L) == 0
            for r in range(_B):
                res = jnp.sum(accbuf[pl.ds(r * _L, _L)])
                idx = jnp.full((_L,), r * chans_per_worker + ci, jnp.int32)
                plsc.store_scatter(resbuf, [idx], jnp.full((_L,), res),
                                   mask=lane0)
            return 0

        lax.fori_loop(0, chans_per_worker, chan_body, 0)

        for r in range(_B):
            pltpu.sync_copy(
                resbuf.at[pl.ds(r * chans_per_worker, chans_per_worker)],
                out_hbm.at[r, pl.ds(c0, chans_per_worker)])

    return body


def kernel(x, weight):
    b, c, n = x.shape
    assert (b, n) == (_B, _N) and weight.shape == (c, n)
    info = plsc.get_sparse_core_info()
    num_workers = info.num_cores * info.num_subcores
    chans_per_worker = c // num_workers
    mesh = plsc.VectorSubcoreMesh(core_axis_name="sc_core",
                                  subcore_axis_name="sc_subcore")
    scratch = [
        pltpu.VMEM((_B * _N,), jnp.float32),
        pltpu.VMEM((_B * _N,), jnp.float32),
        pltpu.VMEM((_B * _PASSES * _NBINS,), jnp.int32),
        pltpu.VMEM((2 * _B * _N,), jnp.float32),
        pltpu.VMEM((2 * _N,), jnp.float32),
        pltpu.VMEM((_B * _L,), jnp.float32),
        pltpu.VMEM((_B * chans_per_worker,), jnp.float32),
        pltpu.SemaphoreType.DMA,
    ]
    k = pl.kernel(
        _fspool_body(num_workers, chans_per_worker, "sc_core", "sc_subcore"),
        out_type=jax.ShapeDtypeStruct((b, c), jnp.float32),
        mesh=mesh,
        scratch_types=scratch,
        compiler_params=pltpu.CompilerParams(needs_layout_passes=False),
    )
    wflip = jnp.flip(weight, axis=1)
    return k(x, wflip)
